# Initial kernel scaffold; baseline (speedup 1.0000x reference)
import jax
import jax.numpy as jnp
from jax.experimental import pallas as pl


def kernel(x, edge_index, batch, W1, b1, cw1, p1, beta1, W2, b2, cw2, p2, beta2, W3, b3, cw3, p3, beta3, lw1, lb1, lw2, lb2, lw3, lb3):
    raise NotImplementedError("write your pallas kernel here")


# speedup vs baseline: 1.2829x; 1.2829x over previous
import functools
import math

import jax
import jax.numpy as jnp
from jax import lax
from jax.experimental import pallas as pl
from jax.experimental.pallas import tpu as pltpu
from jax.experimental.pallas import tpu_sc as plsc

N_EDGES = 320000
RATIO = 0.5

PAD = 128
CH = 128
TILE_CAP = 10240
NCHUNK = TILE_CAP // CH
RING = 2


def _chunk_bounds(E, num_sc=2, tiles=16, unit=320):
    bs = [0]
    half = E // num_sc
    units = half // unit
    per = units // tiles
    extra = units - per * tiles
    for sc in range(num_sc):
        pos = sc * half
        for t in range(tiles):
            pos += (per + (1 if t < extra else 0)) * unit
            bs.append(pos)
    return bs


CHUNK_BOUNDS = _chunk_bounds(N_EDGES)


def _mm_body(x_ref, w_ref, b_ref, o_ref):
    o_ref[...] = jnp.dot(x_ref[...], w_ref[...],
                         preferred_element_type=jnp.float32) + b_ref[...]


def _matmul_bias(x, W, b, blk=512):
    n, kdim = x.shape
    m = W.shape[1]
    npad = (-n) % blk
    xp = jnp.pad(x, ((0, npad), (0, 0)))
    out = pl.pallas_call(
        _mm_body,
        grid=(xp.shape[0] // blk,),
        in_specs=[
            pl.BlockSpec((blk, kdim), lambda i: (i, 0)),
            pl.BlockSpec((kdim, m), lambda i: (0, 0)),
            pl.BlockSpec((m,), lambda i: (0,)),
        ],
        out_specs=pl.BlockSpec((blk, m), lambda i: (i, 0)),
        out_shape=jax.ShapeDtypeStruct((xp.shape[0], m), jnp.float32),
    )(xp, W, b)
    return out[:n]


def _sc_hop_kernel(trash_base,
                   srow_hbm, scol_hbm, y_hbm, direct_hbm, parts_hbm, pids_hbm,
                   rows_v, idx_v, gath_v, stag_v, sid_v, pbuf_v, pid_v,
                   sem_g, sem_o):
    c = lax.axis_index("c")
    s = lax.axis_index("s")
    wid = c * 16 + s

    pltpu.sync_copy(srow_hbm.at[wid], rows_v)
    pltpu.sync_copy(scol_hbm.at[wid], idx_v)
    pltpu.async_copy(y_hbm.at[idx_v.at[0]], gath_v.at[0], sem_g).wait()

    def flush_bank(bank, first):
        @pl.when(first)
        def _():
            for j in range(4):
                pbuf_v[0, pl.ds(16 * j, 16)] = stag_v[0, 0, pl.ds(16 * j, 16)]
            pid_v[pl.ds(0, 1)] = sid_v[0, pl.ds(0, 1)]
            sid_v[0, pl.ds(0, 1)] = jnp.full((1,), trash_base, jnp.int32)
        pltpu.async_copy(stag_v.at[bank], direct_hbm.at[sid_v.at[bank]],
                         sem_o).wait()

    def chunk_body(cix, carry):
        r_cur, nout, flushed, a0, a1, a2, a3 = carry
        b = cix % 2
        nb = (cix + 1) % 2

        @pl.when(cix + 1 < NCHUNK)
        def _():
            pltpu.async_copy(y_hbm.at[idx_v.at[cix + 1]], gath_v.at[nb],
                             sem_g)

        def edge_body(el, ec):
            r_cur, nout, a0, a1, a2, a3 = ec
            r = rows_v[cix, pl.ds(el, 1)][0]
            ch = r != r_cur
            nout = nout + ch.astype(jnp.int32)
            slot = nout & (RING * CH - 1)
            bs = slot // CH
            sl = slot % CH
            g0 = gath_v[b, el, pl.ds(0, 16)]
            g1 = gath_v[b, el, pl.ds(16, 16)]
            g2 = gath_v[b, el, pl.ds(32, 16)]
            g3 = gath_v[b, el, pl.ds(48, 16)]
            a0 = jnp.where(ch, g0, a0 + g0)
            a1 = jnp.where(ch, g1, a1 + g1)
            a2 = jnp.where(ch, g2, a2 + g2)
            a3 = jnp.where(ch, g3, a3 + g3)
            stag_v[bs, sl, pl.ds(0, 16)] = a0
            stag_v[bs, sl, pl.ds(16, 16)] = a1
            stag_v[bs, sl, pl.ds(32, 16)] = a2
            stag_v[bs, sl, pl.ds(48, 16)] = a3
            sid_v[bs, pl.ds(sl, 1)] = jnp.broadcast_to(r, (1,))
            return r, nout, a0, a1, a2, a3

        r_cur, nout, a0, a1, a2, a3 = lax.fori_loop(
            0, CH, edge_body, (r_cur, nout, a0, a1, a2, a3), unroll=4)

        do_flush = (nout - flushed) >= CH

        @pl.when(do_flush)
        def _():
            flush_bank((flushed // CH) % RING, flushed == 0)

        flushed = jnp.where(do_flush, flushed + CH, flushed)

        @pl.when(cix + 1 < NCHUNK)
        def _():
            pltpu.make_async_copy(y_hbm.at[idx_v.at[cix + 1]], gath_v.at[nb],
                                  sem_g).wait()
        return r_cur, nout, flushed, a0, a1, a2, a3

    zero16 = jnp.zeros((16,), jnp.float32)
    r_cur, nout, flushed, a0, a1, a2, a3 = lax.fori_loop(
        0, NCHUNK, chunk_body,
        (jnp.int32(-1), jnp.int32(-1), jnp.int32(0),
         zero16, zero16, zero16, zero16))

    rem = nout - flushed

    @pl.when(rem > 0)
    def _():
        bank = (flushed // CH) % RING
        lane = lax.broadcasted_iota(jnp.int32, (16,), 0)
        for v in range(CH // 16):
            base = v * 16
            cur = sid_v[bank, pl.ds(base, 16)]
            sid_v[bank, pl.ds(base, 16)] = jnp.where(
                (base + lane) < rem, cur, trash_base + base + lane)
        flush_bank(bank, flushed == 0)

    os_ = nout & (RING * CH - 1)
    ob = os_ // CH
    ol = os_ % CH

    @pl.when(nout > 0)
    def _():
        for j in range(4):
            pbuf_v[1, pl.ds(16 * j, 16)] = stag_v[ob, ol, pl.ds(16 * j, 16)]
        pid_v[pl.ds(1, 1)] = jnp.broadcast_to(r_cur, (1,))

    @pl.when(nout == 0)
    def _():
        for j in range(4):
            pbuf_v[0, pl.ds(16 * j, 16)] = stag_v[0, 0, pl.ds(16 * j, 16)]
        pid_v[pl.ds(0, 1)] = jnp.broadcast_to(r_cur, (1,))
        pid_v[pl.ds(1, 1)] = jnp.full((1,), -1, jnp.int32)

    for j in range(2, 8):
        pid_v[pl.ds(j, 1)] = jnp.full((1,), -1, jnp.int32)

    pltpu.sync_copy(pbuf_v, parts_hbm.at[wid])
    pltpu.sync_copy(pid_v, pids_hbm.at[wid])


def _sc_hop(srow_pad, scol_pad, y_pad, nseg):
    trash_base = nseg
    nrows_direct = nseg + CH + 64
    mesh = plsc.VectorSubcoreMesh(core_axis_name="c", subcore_axis_name="s")
    f = pl.kernel(
        functools.partial(_sc_hop_kernel, trash_base),
        mesh=mesh,
        compiler_params=pltpu.CompilerParams(use_tc_tiling_on_sc=False),
        out_type=[
            jax.ShapeDtypeStruct((nrows_direct, 64), jnp.float32),
            jax.ShapeDtypeStruct((32, 2, 64), jnp.float32),
            jax.ShapeDtypeStruct((32, 8), jnp.int32),
        ],
        scratch_types=[
            pltpu.VMEM((NCHUNK, CH), jnp.int32),
            pltpu.VMEM((NCHUNK, CH), jnp.int32),
            pltpu.VMEM((2, CH, 64), jnp.float32),
            pltpu.VMEM((RING, CH, 64), jnp.float32),
            pltpu.VMEM((RING, CH), jnp.int32),
            pltpu.VMEM((2, 64), jnp.float32),
            pltpu.VMEM((8,), jnp.int32),
            pltpu.SemaphoreType.DMA,
            pltpu.SemaphoreType.DMA,
        ],
    )
    return f(srow_pad, scol_pad, y_pad)


def _combine_body(n_real, blk, direct_ref, parts_ref, pids_ref, deg_ref,
                  o_ref):
    i = pl.program_id(0)
    base = i * blk
    rows = base + lax.broadcasted_iota(jnp.int32, (blk, 1), 0)

    pmask = jnp.zeros((blk, 1), jnp.bool_)
    for t in range(32):
        for slot in range(2):
            vid = pids_ref[t, slot]
            pmask = pmask | (rows == vid)
    valid = (rows < n_real) & (deg_ref[...] > 0)
    out = jnp.where(valid & ~pmask, direct_ref[...], 0.0)

    for t in range(32):
        for slot in range(2):
            vid = pids_ref[t, slot]
            cond = (vid >= base) & (vid < base + blk) & (vid < n_real)
            part = parts_ref[t, slot, :]
            onehot = (rows == vid).astype(jnp.float32)
            out = jnp.where(cond, out + onehot * part[None, :], out)
    o_ref[...] = out


def _combine(direct, parts, pids, deg, n_real, n_out, blk=512):
    npad = (-n_out) % blk
    total = n_out + npad
    dpad = jnp.pad(direct[:n_out], ((0, npad), (0, 0)))
    degp = jnp.pad(deg[:n_out], (0, npad)).reshape(total, 1)
    out = pl.pallas_call(
        functools.partial(_combine_body, n_real, blk),
        grid=(total // blk,),
        in_specs=[
            pl.BlockSpec((blk, 64), lambda i: (i, 0)),
            pl.BlockSpec((32, 2, 64), lambda i: (0, 0, 0)),
            pl.BlockSpec(memory_space=pltpu.SMEM),
            pl.BlockSpec((blk, 1), lambda i: (i, 0)),
        ],
        out_specs=pl.BlockSpec((blk, 64), lambda i: (i, 0)),
        out_shape=jax.ShapeDtypeStruct((total, 64), jnp.float32),
    )(dpad, parts, pids, degp)
    return out[:n_out]


def _spmm_ref(row, col, ew, y):
    n = y.shape[0]
    yp = jnp.concatenate([y, jnp.zeros((1, y.shape[1]), y.dtype)], axis=0)
    msg = ew[:, None] * yp[col]
    return jax.ops.segment_sum(msg, row, num_segments=n + 1)[:n]


def _m_apply_ref(row, col, ew, cw, y, transpose=False):
    r, c = (col, row) if transpose else (row, col)
    out = cw[0] * y
    tmp = y
    for i in range(1, cw.shape[0]):
        tmp = _spmm_ref(r, c, ew, tmp)
        out = out + cw[i] * tmp
    return out


def _m_apply_sc(srow_pad, scol_pad, deg, cw, y, n_real):
    nseg = n_real + 1
    out = cw[0] * y
    tmp_pad = jnp.concatenate([y, jnp.zeros((PAD, 64), jnp.float32)], axis=0)
    for i in range(1, cw.shape[0]):
        direct, parts, pids = _sc_hop(srow_pad, scol_pad, tmp_pad, nseg)
        tmp_pad = _combine(direct, parts, pids, deg, n_real, n_real + PAD)
        out = out + cw[i] * tmp_pad[:n_real]
    return out


def _build_layout(srow, scol, pad_col_base):
    rows_t, cols_t = [], []
    for t in range(32):
        lo, hi = CHUNK_BOUNDS[t], CHUNK_BOUNDS[t + 1]
        m = hi - lo
        r = srow[lo:hi]
        c = scol[lo:hi]
        if m < TILE_CAP:
            padr = jnp.broadcast_to(r[m - 1], (TILE_CAP - m,))
            padc = pad_col_base + (jnp.arange(TILE_CAP - m,
                                              dtype=jnp.int32) % PAD)
            r = jnp.concatenate([r, padr])
            c = jnp.concatenate([c, padc])
        rows_t.append(r)
        cols_t.append(c)
    srow_pad = jnp.stack(rows_t).reshape(32, NCHUNK, CH)
    scol_pad = jnp.stack(cols_t).reshape(32, NCHUNK, CH)
    return srow_pad, scol_pad


def _conv_layer(x, row, col, ew, W, b, cw, srow_pad, scol_pad, deg):
    n = x.shape[0]
    h = _matmul_bias(x, W, b)
    ones = jnp.ones((n, 1), x.dtype)
    d = _m_apply_ref(row, col, ew, cw, ones)[:, 0]
    d = jnp.clip(d, 1e-10, None)
    dinv = d ** -0.5
    out = dinv[:, None] * _m_apply_sc(srow_pad, scol_pad, deg, cw,
                                      dinv[:, None] * h, n)
    colsum = dinv * _m_apply_ref(row, col, ew, cw, dinv[:, None],
                                 transpose=True)[:, 0]
    return out, colsum


def _pan_pool(x, row, col, ew, batch, colsum, p, beta, ratio):
    n = x.shape[0]
    k = int(math.ceil(ratio * n))
    score = jnp.tanh(beta[0] * (x @ p) + beta[1] * colsum)
    score_perm, perm = jax.lax.top_k(score, k)
    x_new = x[perm] * score_perm[:, None]
    mapping = jnp.full((n + 1,), k, dtype=row.dtype).at[perm].set(
        jnp.arange(k, dtype=row.dtype))
    r2 = mapping[row]
    c2 = mapping[col]
    ew_new = ew * ((r2 < k) & (c2 < k)).astype(ew.dtype)
    return x_new, r2, c2, ew_new, batch[perm], perm, score_perm


def kernel(x, edge_index, batch, W1, b1, cw1, p1, beta1, W2, b2, cw2, p2,
           beta2, W3, b3, cw3, p3, beta3, lw1, lb1, lw2, lb2, lw3, lb3):
    row, col = edge_index[0], edge_index[1]
    ew = jnp.ones((row.shape[0],), jnp.float32)
    n = x.shape[0]
    eid = jnp.arange(N_EDGES, dtype=jnp.int32)

    perms = []
    h = x
    for W_, b_, cw_, p_, beta_ in ((W1, b1, cw1, p1, beta1),
                                   (W2, b2, cw2, p2, beta2),
                                   (W3, b3, cw3, p3, beta3)):
        k = int(math.ceil(RATIO * n))
        alive = ew > 0
        scol_eff = jnp.where(alive, col, n + (eid % PAD)).astype(jnp.int32)
        order = jnp.argsort(row, stable=True)
        srow = row[order].astype(jnp.int32)
        scol = scol_eff[order]
        srow_pad, scol_pad = _build_layout(srow, scol, n)
        deg = jax.ops.segment_sum(jnp.ones((N_EDGES,), jnp.int32), row,
                                  num_segments=n + PAD)

        h, cs = _conv_layer(h, row, col, ew, W_, b_, cw_, srow_pad, scol_pad,
                            deg)
        h, row, col, ew, batch, perm, _ = _pan_pool(
            h, row, col, ew, batch, cs, p_, beta_, RATIO)
        perms.append(perm)
        n = k

    num_graphs = 1
    s = jax.ops.segment_sum(h, batch, num_segments=num_graphs)
    cnt = jax.ops.segment_sum(jnp.ones((h.shape[0],), h.dtype), batch,
                              num_segments=num_graphs)
    mean = s / jnp.clip(cnt, 1.0, None)[:, None]
    z = jax.nn.relu(mean @ lw1 + lb1)
    z = jax.nn.relu(z @ lw2 + lb2)
    out = jax.nn.log_softmax(z @ lw3 + lb3, axis=-1)
    return (out, perms[0], perms[1], perms[2])


# speedup vs baseline: 5.2677x; 4.1059x over previous
import functools
import math

import jax
import jax.numpy as jnp
from jax import lax
from jax.experimental import pallas as pl
from jax.experimental.pallas import tpu as pltpu
from jax.experimental.pallas import tpu_sc as plsc

N_EDGES = 320000
RATIO = 0.5

PAD = 128
CH = 128
TILE_CAP = 10240
NCHUNK = TILE_CAP // CH
RING = 2


def _chunk_bounds(E, num_sc=2, tiles=16, unit=320):
    bs = [0]
    half = E // num_sc
    units = half // unit
    per = units // tiles
    extra = units - per * tiles
    for sc in range(num_sc):
        pos = sc * half
        for t in range(tiles):
            pos += (per + (1 if t < extra else 0)) * unit
            bs.append(pos)
    return bs


CHUNK_BOUNDS = _chunk_bounds(N_EDGES)


def _mm_body(x_ref, w_ref, b_ref, o_ref):
    o_ref[...] = jnp.dot(x_ref[...], w_ref[...],
                         preferred_element_type=jnp.float32) + b_ref[...]


def _matmul_bias(x, W, b, blk=512):
    n, kdim = x.shape
    m = W.shape[1]
    npad = (-n) % blk
    xp = jnp.pad(x, ((0, npad), (0, 0)))
    out = pl.pallas_call(
        _mm_body,
        grid=(xp.shape[0] // blk,),
        in_specs=[
            pl.BlockSpec((blk, kdim), lambda i: (i, 0)),
            pl.BlockSpec((kdim, m), lambda i: (0, 0)),
            pl.BlockSpec((m,), lambda i: (0,)),
        ],
        out_specs=pl.BlockSpec((blk, m), lambda i: (i, 0)),
        out_shape=jax.ShapeDtypeStruct((xp.shape[0], m), jnp.float32),
    )(xp, W, b)
    return out[:n]


def _sc_hop_kernel(trash_base,
                   srow_hbm, scol_hbm, y_hbm, direct_hbm, parts_hbm, pids_hbm,
                   rows_v, idx_v, gath_v, stag_v, sid_v, pbuf_v, pid_v,
                   sem_g, sem_o):
    c = lax.axis_index("c")
    s = lax.axis_index("s")
    wid = c * 16 + s

    pltpu.sync_copy(srow_hbm.at[wid], rows_v)
    pltpu.sync_copy(scol_hbm.at[wid], idx_v)
    pltpu.async_copy(y_hbm.at[idx_v.at[0]], gath_v.at[0], sem_g).wait()

    def flush_bank(bank, first):
        @pl.when(first)
        def _():
            for j in range(4):
                pbuf_v[0, pl.ds(16 * j, 16)] = stag_v[0, 0, pl.ds(16 * j, 16)]
            pid_v[pl.ds(0, 1)] = sid_v[0, pl.ds(0, 1)]
            sid_v[0, pl.ds(0, 1)] = jnp.full((1,), trash_base, jnp.int32)
        pltpu.async_copy(stag_v.at[bank], direct_hbm.at[sid_v.at[bank]],
                         sem_o).wait()

    def chunk_body(cix, carry):
        r_cur, nout, flushed, a0, a1, a2, a3 = carry
        b = cix % 2
        nb = (cix + 1) % 2

        @pl.when(cix + 1 < NCHUNK)
        def _():
            pltpu.async_copy(y_hbm.at[idx_v.at[cix + 1]], gath_v.at[nb],
                             sem_g)

        def edge_body(el, ec):
            r_cur, nout, a0, a1, a2, a3 = ec
            r = rows_v[cix, pl.ds(el, 1)][0]
            ch = r != r_cur
            nout = nout + ch.astype(jnp.int32)
            slot = nout & (RING * CH - 1)
            bs = slot // CH
            sl = slot % CH
            g0 = gath_v[b, el, pl.ds(0, 16)]
            g1 = gath_v[b, el, pl.ds(16, 16)]
            g2 = gath_v[b, el, pl.ds(32, 16)]
            g3 = gath_v[b, el, pl.ds(48, 16)]
            a0 = jnp.where(ch, g0, a0 + g0)
            a1 = jnp.where(ch, g1, a1 + g1)
            a2 = jnp.where(ch, g2, a2 + g2)
            a3 = jnp.where(ch, g3, a3 + g3)
            stag_v[bs, sl, pl.ds(0, 16)] = a0
            stag_v[bs, sl, pl.ds(16, 16)] = a1
            stag_v[bs, sl, pl.ds(32, 16)] = a2
            stag_v[bs, sl, pl.ds(48, 16)] = a3
            sid_v[bs, pl.ds(sl, 1)] = jnp.broadcast_to(r, (1,))
            return r, nout, a0, a1, a2, a3

        r_cur, nout, a0, a1, a2, a3 = lax.fori_loop(
            0, CH, edge_body, (r_cur, nout, a0, a1, a2, a3), unroll=4)

        do_flush = (nout - flushed) >= CH

        @pl.when(do_flush)
        def _():
            flush_bank((flushed // CH) % RING, flushed == 0)

        flushed = jnp.where(do_flush, flushed + CH, flushed)

        @pl.when(cix + 1 < NCHUNK)
        def _():
            pltpu.make_async_copy(y_hbm.at[idx_v.at[cix + 1]], gath_v.at[nb],
                                  sem_g).wait()
        return r_cur, nout, flushed, a0, a1, a2, a3

    zero16 = jnp.zeros((16,), jnp.float32)
    r_cur, nout, flushed, a0, a1, a2, a3 = lax.fori_loop(
        0, NCHUNK, chunk_body,
        (jnp.int32(-1), jnp.int32(-1), jnp.int32(0),
         zero16, zero16, zero16, zero16))

    rem = nout - flushed

    @pl.when(rem > 0)
    def _():
        bank = (flushed // CH) % RING
        lane = lax.broadcasted_iota(jnp.int32, (16,), 0)
        for v in range(CH // 16):
            base = v * 16
            cur = sid_v[bank, pl.ds(base, 16)]
            sid_v[bank, pl.ds(base, 16)] = jnp.where(
                (base + lane) < rem, cur, trash_base + base + lane)
        flush_bank(bank, flushed == 0)

    os_ = nout & (RING * CH - 1)
    ob = os_ // CH
    ol = os_ % CH

    @pl.when(nout > 0)
    def _():
        for j in range(4):
            pbuf_v[1, pl.ds(16 * j, 16)] = stag_v[ob, ol, pl.ds(16 * j, 16)]
        pid_v[pl.ds(1, 1)] = jnp.broadcast_to(r_cur, (1,))

    @pl.when(nout == 0)
    def _():
        for j in range(4):
            pbuf_v[0, pl.ds(16 * j, 16)] = stag_v[0, 0, pl.ds(16 * j, 16)]
        pid_v[pl.ds(0, 1)] = jnp.broadcast_to(r_cur, (1,))
        pid_v[pl.ds(1, 1)] = jnp.full((1,), -1, jnp.int32)

    for j in range(2, 8):
        pid_v[pl.ds(j, 1)] = jnp.full((1,), -1, jnp.int32)

    pltpu.sync_copy(pbuf_v, parts_hbm.at[wid])
    pltpu.sync_copy(pid_v, pids_hbm.at[wid])


def _sc_hop(srow_pad, scol_pad, y_pad, nseg):
    trash_base = nseg
    nrows_direct = nseg + CH + 64
    mesh = plsc.VectorSubcoreMesh(core_axis_name="c", subcore_axis_name="s")
    f = pl.kernel(
        functools.partial(_sc_hop_kernel, trash_base),
        mesh=mesh,
        compiler_params=pltpu.CompilerParams(use_tc_tiling_on_sc=False),
        out_type=[
            jax.ShapeDtypeStruct((nrows_direct, 64), jnp.float32),
            jax.ShapeDtypeStruct((32, 2, 64), jnp.float32),
            jax.ShapeDtypeStruct((32, 8), jnp.int32),
        ],
        scratch_types=[
            pltpu.VMEM((NCHUNK, CH), jnp.int32),
            pltpu.VMEM((NCHUNK, CH), jnp.int32),
            pltpu.VMEM((2, CH, 64), jnp.float32),
            pltpu.VMEM((RING, CH, 64), jnp.float32),
            pltpu.VMEM((RING, CH), jnp.int32),
            pltpu.VMEM((2, 64), jnp.float32),
            pltpu.VMEM((8,), jnp.int32),
            pltpu.SemaphoreType.DMA,
            pltpu.SemaphoreType.DMA,
        ],
    )
    return f(srow_pad, scol_pad, y_pad)


def _combine_body(n_real, blk, direct_ref, parts_ref, pids_ref, deg_ref,
                  o_ref):
    i = pl.program_id(0)
    base = i * blk
    rows = base + lax.broadcasted_iota(jnp.int32, (blk, 1), 0)

    pmask = jnp.zeros((blk, 1), jnp.bool_)
    for t in range(32):
        for slot in range(2):
            vid = pids_ref[t, slot]
            pmask = pmask | (rows == vid)
    valid = (rows < n_real) & (deg_ref[...] > 0)
    out = jnp.where(valid & ~pmask, direct_ref[...], 0.0)

    for t in range(32):
        for slot in range(2):
            vid = pids_ref[t, slot]
            cond = (vid >= base) & (vid < base + blk) & (vid < n_real)
            part = parts_ref[t, slot, :]
            onehot = (rows == vid).astype(jnp.float32)
            out = jnp.where(cond, out + onehot * part[None, :], out)
    o_ref[...] = out


def _combine(direct, parts, pids, deg, n_real, n_out, blk=512):
    npad = (-n_out) % blk
    total = n_out + npad
    dpad = jnp.pad(direct[:n_out], ((0, npad), (0, 0)))
    degp = jnp.pad(deg[:n_out], (0, npad)).reshape(total, 1)
    out = pl.pallas_call(
        functools.partial(_combine_body, n_real, blk),
        grid=(total // blk,),
        in_specs=[
            pl.BlockSpec((blk, 64), lambda i: (i, 0)),
            pl.BlockSpec((32, 2, 64), lambda i: (0, 0, 0)),
            pl.BlockSpec(memory_space=pltpu.SMEM),
            pl.BlockSpec((blk, 1), lambda i: (i, 0)),
        ],
        out_specs=pl.BlockSpec((blk, 64), lambda i: (i, 0)),
        out_shape=jax.ShapeDtypeStruct((total, 64), jnp.float32),
    )(dpad, parts, pids, degp)
    return out[:n_out]


EPT = N_EDGES // 32


def _sc_gather_kernel(tbl_hbm, idx_hbm, out_hbm, tbl_v, idx_v, out_v, sem):
    c = lax.axis_index("c")
    s = lax.axis_index("s")
    wid = c * 16 + s
    pltpu.sync_copy(tbl_hbm, tbl_v)
    pltpu.sync_copy(idx_hbm.at[wid], idx_v)

    def body(e, carry):
        ix = idx_v[pl.ds(e, 1)][0]
        out_v[pl.ds(e, 1)] = tbl_v[pl.ds(ix, 1)]
        return carry

    lax.fori_loop(0, EPT, body, jnp.int32(0), unroll=8)
    pltpu.sync_copy(out_v, out_hbm.at[wid])


def _sc_gather(table, idx):
    dt = table.dtype
    mesh = plsc.VectorSubcoreMesh(core_axis_name="c", subcore_axis_name="s")
    f = pl.kernel(
        _sc_gather_kernel,
        mesh=mesh,
        compiler_params=pltpu.CompilerParams(use_tc_tiling_on_sc=False),
        out_type=jax.ShapeDtypeStruct((32, EPT), dt),
        scratch_types=[
            pltpu.VMEM((table.shape[0],), dt),
            pltpu.VMEM((EPT,), jnp.int32),
            pltpu.VMEM((EPT,), dt),
            pltpu.SemaphoreType.DMA,
        ],
    )
    return f(table, idx.reshape(32, EPT)).reshape(N_EDGES)


def _pad_to(v, mult):
    p = (-v.shape[0]) % mult
    return jnp.concatenate([v, jnp.zeros((p,), v.dtype)]) if p else v


def _spmm_ref(row, col, ew, y):
    n = y.shape[0]
    tbl = _pad_to(jnp.concatenate([y[:, 0], jnp.zeros((1,), y.dtype)]), 256)
    msg = (ew * _sc_gather(tbl, col))[:, None]
    return jax.ops.segment_sum(msg, row, num_segments=n + 1)[:n]


def _m_apply_ref(row, col, ew, cw, y, transpose=False):
    r, c = (col, row) if transpose else (row, col)
    out = cw[0] * y
    tmp = y
    for i in range(1, cw.shape[0]):
        tmp = _spmm_ref(r, c, ew, tmp)
        out = out + cw[i] * tmp
    return out


def _m_apply_sc(srow_pad, scol_pad, deg, cw, y, n_real):
    nseg = n_real + 1
    out = cw[0] * y
    tmp_pad = jnp.concatenate([y, jnp.zeros((PAD, 64), jnp.float32)], axis=0)
    for i in range(1, cw.shape[0]):
        direct, parts, pids = _sc_hop(srow_pad, scol_pad, tmp_pad, nseg)
        tmp_pad = _combine(direct, parts, pids, deg, n_real, n_real + PAD)
        out = out + cw[i] * tmp_pad[:n_real]
    return out


def _build_layout(srow, scol, pad_col_base):
    rows_t, cols_t = [], []
    for t in range(32):
        lo, hi = CHUNK_BOUNDS[t], CHUNK_BOUNDS[t + 1]
        m = hi - lo
        r = srow[lo:hi]
        c = scol[lo:hi]
        if m < TILE_CAP:
            padr = jnp.broadcast_to(r[m - 1], (TILE_CAP - m,))
            padc = pad_col_base + (jnp.arange(TILE_CAP - m,
                                              dtype=jnp.int32) % PAD)
            r = jnp.concatenate([r, padr])
            c = jnp.concatenate([c, padc])
        rows_t.append(r)
        cols_t.append(c)
    srow_pad = jnp.stack(rows_t).reshape(32, NCHUNK, CH)
    scol_pad = jnp.stack(cols_t).reshape(32, NCHUNK, CH)
    return srow_pad, scol_pad


def _conv_layer(x, row, col, ew, W, b, cw, srow_pad, scol_pad, deg):
    n = x.shape[0]
    h = _matmul_bias(x, W, b)
    ones = jnp.ones((n, 1), x.dtype)
    d = _m_apply_ref(row, col, ew, cw, ones)[:, 0]
    d = jnp.clip(d, 1e-10, None)
    dinv = d ** -0.5
    out = dinv[:, None] * _m_apply_sc(srow_pad, scol_pad, deg, cw,
                                      dinv[:, None] * h, n)
    colsum = dinv * _m_apply_ref(row, col, ew, cw, dinv[:, None],
                                 transpose=True)[:, 0]
    return out, colsum


def _pan_pool(x, row, col, ew, batch, colsum, p, beta, ratio):
    n = x.shape[0]
    k = int(math.ceil(ratio * n))
    score = jnp.tanh(beta[0] * (x @ p) + beta[1] * colsum)
    score_perm, perm = jax.lax.top_k(score, k)
    x_new = x[perm] * score_perm[:, None]
    mapping = jnp.full((n + 1,), k, dtype=row.dtype).at[perm].set(
        jnp.arange(k, dtype=row.dtype))
    mpad = _pad_to(mapping.astype(jnp.int32), 256)
    r2 = _sc_gather(mpad, row)
    c2 = _sc_gather(mpad, col)
    ew_new = ew * ((r2 < k) & (c2 < k)).astype(ew.dtype)
    return x_new, r2, c2, ew_new, batch[perm], perm, score_perm


def kernel(x, edge_index, batch, W1, b1, cw1, p1, beta1, W2, b2, cw2, p2,
           beta2, W3, b3, cw3, p3, beta3, lw1, lb1, lw2, lb2, lw3, lb3):
    row, col = edge_index[0], edge_index[1]
    ew = jnp.ones((row.shape[0],), jnp.float32)
    n = x.shape[0]
    eid = jnp.arange(N_EDGES, dtype=jnp.int32)

    perms = []
    h = x
    for W_, b_, cw_, p_, beta_ in ((W1, b1, cw1, p1, beta1),
                                   (W2, b2, cw2, p2, beta2),
                                   (W3, b3, cw3, p3, beta3)):
        k = int(math.ceil(RATIO * n))
        alive = ew > 0
        scol_eff = jnp.where(alive, col, n + (eid % PAD)).astype(jnp.int32)
        order = jnp.argsort(row, stable=True)
        srow = row[order].astype(jnp.int32)
        scol = scol_eff[order]
        srow_pad, scol_pad = _build_layout(srow, scol, n)
        deg = jax.ops.segment_sum(jnp.ones((N_EDGES,), jnp.int32), row,
                                  num_segments=n + PAD)

        h, cs = _conv_layer(h, row, col, ew, W_, b_, cw_, srow_pad, scol_pad,
                            deg)
        h, row, col, ew, batch, perm, _ = _pan_pool(
            h, row, col, ew, batch, cs, p_, beta_, RATIO)
        perms.append(perm)
        n = k

    num_graphs = 1
    s = jax.ops.segment_sum(h, batch, num_segments=num_graphs)
    cnt = jax.ops.segment_sum(jnp.ones((h.shape[0],), h.dtype), batch,
                              num_segments=num_graphs)
    mean = s / jnp.clip(cnt, 1.0, None)[:, None]
    z = jax.nn.relu(mean @ lw1 + lb1)
    z = jax.nn.relu(z @ lw2 + lb2)
    out = jax.nn.log_softmax(z @ lw3 + lb3, axis=-1)
    return (out, perms[0], perms[1], perms[2])
